# baseline (speedup 1.0000x reference)
import functools

import jax
import jax.numpy as jnp
from jax import lax
from jax.experimental import pallas as pl
from jax.experimental.pallas import tpu as pltpu
from jax.experimental.pallas import tpu_sc as plsc

MAX_DEG = 512
ROWS_PAD = 520
D_IN = 128
D_OUT = 64
L = 16
CH = 128


def _project_body(t1_ref, t2_ref, w_ref, b_ref, p_ref):
    w = w_ref[...]
    dn = (((1,), (1,)), ((), ()))
    p_ref[:, 0:D_OUT] = lax.dot_general(t1_ref[...], w, dn,
                                        preferred_element_type=jnp.float32)
    p_ref[:, D_OUT:D_IN] = lax.dot_general(t2_ref[...], w, dn,
                                           preferred_element_type=jnp.float32
                                           ) + b_ref[...]


def _make_sc_kernel(n_total):
    nc, ns = 2, 16
    nw = nc * ns
    n_chunks = -(-n_total // CH)
    full_rounds = (n_chunks - 1) // nw
    rem = n_chunks - 1 - full_rounds * nw
    assert n_total % 8 == 0

    mesh = plsc.VectorSubcoreMesh(core_axis_name="c", subcore_axis_name="s",
                                  num_cores=nc, num_subcores=ns)

    @functools.partial(
        pl.kernel,
        out_type=jax.ShapeDtypeStruct((D_OUT, n_chunks * CH), jnp.float32),
        mesh=mesh,
        scratch_types=[
            pltpu.VMEM((ROWS_PAD, D_IN), jnp.float32),
            pltpu.VMEM((CH,), jnp.int32),
            pltpu.VMEM((CH,), jnp.int32),
            pltpu.VMEM((D_OUT, CH + 1), jnp.float32),
            pltpu.SemaphoreType.DMA,
        ],
        compiler_params=pltpu.CompilerParams(needs_layout_passes=False),
    )
    def sc_kernel(tp_hbm, ind_hbm, outd_hbm, out_hbm,
                  tbl_v, idx1_v, idx2_v, outb_v, sem_tbl):
        wid = lax.axis_index("s") * nc + lax.axis_index("c")
        iota = jax.lax.iota(jnp.int32, L)

        def compute_chunk():
            @plsc.parallel_loop(0, CH // L, 1, unroll=1)
            def _(g):
                gs = pl.ds(g * L, L)
                i1v = jnp.clip(idx1_v[gs], 0, MAX_DEG)
                i2v = jnp.clip(idx2_v[gs], 0, MAX_DEG)
                nbase = g * L
                for l in range(L):
                    i1 = i1v[l]
                    i2 = i2v[l]
                    nv = jnp.full((L,), nbase + l, jnp.int32)
                    for d0 in range(0, D_OUT, L):
                        v = tbl_v[i1, pl.ds(d0, L)] + \
                            tbl_v[i2, pl.ds(D_OUT + d0, L)]
                        plsc.store_scatter(outb_v, [iota + d0, nv], v)

        def do_chunk(start):
            s = pl.ds(start, CH)
            pltpu.sync_copy(ind_hbm.at[s], idx1_v)
            pltpu.sync_copy(outd_hbm.at[s], idx2_v)
            compute_chunk()
            pltpu.sync_copy(outb_v.at[:, pl.ds(0, CH)], out_hbm.at[:, s])

        pltpu.async_copy(tp_hbm, tbl_v, sem_tbl).wait()

        def round_body(k, carry):
            do_chunk(pl.multiple_of((wid + k * nw) * CH, CH))
            return carry

        lax.fori_loop(0, full_rounds, round_body, 0)

        if rem >= 0:

            @pl.when(wid <= rem)
            def _():
                do_chunk(pl.multiple_of((full_rounds * nw + wid) * CH, CH))

    return sc_kernel


def kernel(in_degree, out_degree, table1, table2, W, b):
    n_total = in_degree.shape[0]
    pad = ROWS_PAD - table1.shape[0]
    t1 = jnp.pad(table1, ((0, pad), (0, 0)))
    t2 = jnp.pad(table2, ((0, pad), (0, 0)))
    b2 = b.reshape(1, D_OUT)

    tp = pl.pallas_call(
        _project_body,
        out_shape=jax.ShapeDtypeStruct((ROWS_PAD, D_IN), jnp.float32),
    )(t1, t2, W, b2)

    n_pad = -(-n_total // CH) * CH - n_total
    sc_kernel = _make_sc_kernel(n_total)
    out_t = sc_kernel(tp,
                      jnp.pad(in_degree.astype(jnp.int32), (0, n_pad)),
                      jnp.pad(out_degree.astype(jnp.int32), (0, n_pad)))
    return out_t[:, :n_total].T
